# Initial kernel scaffold; baseline (speedup 1.0000x reference)
import functools

import jax
import jax.numpy as jnp
from jax import lax
from jax.experimental import pallas as pl
from jax.experimental.pallas import tpu as pltpu
from jax.experimental.pallas import tpu_sc as plsc

N_USERS = 5000
N_ITEMS = 5000
NT = N_USERS + N_ITEMS
D = 128
E = 320000
N_LAYERS = 3
EPS = 1e-7

NC = 2
NS = 16
NW = NC * NS

C = 125
CH1 = E // (NW * C)
CH0 = E // (NS * C)
NPAD = 10240
NODES_PER_TILE = NPAD // NS
ROWS_PER_TILE = NT // NS

_mesh = plsc.VectorSubcoreMesh(core_axis_name="c", subcore_axis_name="s")


def _zero_rows(ref, b, nrows):
    zv = jnp.zeros((16,), jnp.float32)

    def body(i, _):
        r = i // (D // 16)
        col = (i % (D // 16)) * 16
        ref[b, r, pl.ds(col, 16)] = zv
        return 0

    lax.fori_loop(0, nrows * (D // 16), body, 0)


@functools.partial(
    pl.kernel,
    out_type=(
        jax.ShapeDtypeStruct((NPAD,), jnp.float32),
        jax.ShapeDtypeStruct((NPAD,), jnp.float32),
        jax.ShapeDtypeStruct((NPAD,), jnp.float32),
    ),
    mesh=_mesh,
    scratch_types=[
        pltpu.VMEM((CH0, C), jnp.int32),
        pltpu.VMEM((C, 16), jnp.float32),
        pltpu.VMEM((128, 16), jnp.float32),
        pltpu.VMEM((NODES_PER_TILE, 16), jnp.float32),
        pltpu.VMEM((NODES_PER_TILE,), jnp.float32),
        pltpu.VMEM((NODES_PER_TILE,), jnp.float32),
        pltpu.VMEM((NODES_PER_TILE,), jnp.float32),
        pltpu.VMEM_SHARED((NPAD, 16), jnp.float32),
    ],
)
def _k0_degrees(dst_hbm, dinv_hbm, dinv2_hbm, sq4_hbm,
                dst_v, ones_v, zrow_v, deg_v, dinv_v, dinv2_v, sq4_v, deg_sp):
    cid = lax.axis_index("c")
    sid = lax.axis_index("s")

    @pl.when(cid == 0)
    def _work():
        lane = lax.iota(jnp.int32, 16)
        one_row = jnp.where(lane == 0, 1.0, 0.0).astype(jnp.float32)
        zrow = jnp.zeros((16,), jnp.float32)

        def init_rows(i, _):
            ones_v[i, :] = one_row
            return 0
        lax.fori_loop(0, C, init_rows, 0)

        def init_z(i, _):
            zrow_v[i, :] = zrow
            return 0
        lax.fori_loop(0, 128, init_z, 0)

        base = sid * NODES_PER_TILE
        for k in range(NODES_PER_TILE // 128):
            pltpu.sync_copy(zrow_v, deg_sp.at[pl.ds(base + k * 128, 128)])
        plsc.subcore_barrier()

        pltpu.sync_copy(dst_hbm.at[sid], dst_v)

        def hist(j, _):
            pltpu.sync_copy(ones_v, deg_sp.at[dst_v.at[j]], add=True)
            return 0
        lax.fori_loop(0, CH0, hist, 0)
        plsc.subcore_barrier()

        pltpu.sync_copy(deg_sp.at[pl.ds(base, NODES_PER_TILE)], deg_v)
        zeros_i = jnp.zeros((16,), jnp.int32)

        def scales(g, _):
            rows = lane + g * 16
            deg = plsc.load_gather(deg_v, [rows, zeros_i]) + EPS
            bits = plsc.bitcast(deg, jnp.int32)
            bits = 0x5F3759DF - lax.shift_right_arithmetic(bits, 1)
            y = plsc.bitcast(bits, jnp.float32)
            for _ in range(3):
                y = y * (1.5 - 0.5 * deg * y * y)
            dinv_v[pl.ds(g * 16, 16)] = y
            dinv2_v[pl.ds(g * 16, 16)] = y * y
            sq4_v[pl.ds(g * 16, 16)] = 0.25 * deg * y
            return 0
        lax.fori_loop(0, NODES_PER_TILE // 16, scales, 0)

        pltpu.sync_copy(dinv_v, dinv_hbm.at[pl.ds(base, NODES_PER_TILE)])
        pltpu.sync_copy(dinv2_v, dinv2_hbm.at[pl.ds(base, NODES_PER_TILE)])
        pltpu.sync_copy(sq4_v, sq4_hbm.at[pl.ds(base, NODES_PER_TILE)])


@functools.partial(
    pl.kernel,
    out_type=jax.ShapeDtypeStruct((NC, NT, D), jnp.float32),
    mesh=_mesh,
    scratch_types=[
        pltpu.VMEM((CH1, C), jnp.int32),
        pltpu.VMEM((CH1, C), jnp.int32),
        pltpu.VMEM((2, C, D), jnp.float32),
        pltpu.VMEM_SHARED((NT, D), jnp.float32),
        pltpu.SemaphoreType.DMA,
        pltpu.SemaphoreType.DMA,
    ],
)
def _k1_propagate(z_hbm, src_hbm, dst_hbm, out_hbm,
                  src_v, dst_v, rows_v, acc_sp, sem0, sem1):
    cid = lax.axis_index("c")
    sid = lax.axis_index("s")
    wid = cid * NS + sid

    _zero_rows(rows_v, 0, C)
    base = sid * ROWS_PER_TILE
    for k in range(ROWS_PER_TILE // C):
        pltpu.sync_copy(rows_v.at[0], acc_sp.at[pl.ds(base + k * C, C)])
    plsc.subcore_barrier()

    pltpu.sync_copy(src_hbm.at[wid], src_v)
    pltpu.sync_copy(dst_hbm.at[wid], dst_v)

    sems = (sem0, sem1)
    pltpu.async_copy(z_hbm.at[src_v.at[0]], rows_v.at[0], sem0)

    def body(jj, _):
        for b in range(2):
            j = jj * 2 + b
            pltpu.make_async_copy(
                z_hbm.at[src_v.at[j]], rows_v.at[b], sems[b]).wait()

            @pl.when(j + 1 < CH1)
            def _issue():
                pltpu.async_copy(
                    z_hbm.at[src_v.at[j + 1]], rows_v.at[1 - b], sems[1 - b])

            pltpu.sync_copy(rows_v.at[b], acc_sp.at[dst_v.at[j]], add=True)
        return 0

    lax.fori_loop(0, CH1 // 2, body, 0)
    plsc.subcore_barrier()

    pltpu.sync_copy(acc_sp.at[pl.ds(base, ROWS_PER_TILE)],
                    out_hbm.at[cid, pl.ds(base, ROWS_PER_TILE)])


_BR = 1000
_row_spec = pl.BlockSpec((_BR, D), lambda i: (i, 0))
_col_spec = pl.BlockSpec((_BR, 1), lambda i: (i, 0))


def _tc_scale_body(x_ref, s_ref, o_ref):
    o_ref[...] = x_ref[...] * s_ref[...]


_tc_scale = pl.pallas_call(
    _tc_scale_body,
    grid=(NT // _BR,),
    in_specs=[_row_spec, _col_spec],
    out_specs=_row_spec,
    out_shape=jax.ShapeDtypeStruct((NT, D), jnp.float32),
)


def _tc_combine_body(p0_ref, p1_ref, s_ref, o_ref):
    o_ref[...] = (p0_ref[...] + p1_ref[...]) * s_ref[...]


_tc_combine = pl.pallas_call(
    _tc_combine_body,
    grid=(NT // _BR,),
    in_specs=[_row_spec, _row_spec, _col_spec],
    out_specs=_row_spec,
    out_shape=jax.ShapeDtypeStruct((NT, D), jnp.float32),
)


def _tc_final_body(z0_ref, z1_ref, z2_ref, p0_ref, p1_ref, d2_ref, s4_ref,
                   o_ref):
    z3 = (p0_ref[...] + p1_ref[...]) * d2_ref[...]
    o_ref[...] = (z0_ref[...] + z1_ref[...] + z2_ref[...] + z3) * s4_ref[...]


_tc_final = pl.pallas_call(
    _tc_final_body,
    grid=(NT // _BR,),
    in_specs=[_row_spec, _row_spec, _row_spec, _row_spec, _row_spec,
              _col_spec, _col_spec],
    out_specs=_row_spec,
    out_shape=jax.ShapeDtypeStruct((NT, D), jnp.float32),
)


def kernel(user_emb, item_emb, edge_index):
    src = edge_index[0]
    dst = edge_index[1]

    dst_hist = dst.reshape(NS, CH0, C)
    src_prop = src.reshape(NW, CH1, C)
    dst_prop = dst.reshape(NW, CH1, C)

    dinv, dinv2, sq4 = _k0_degrees(dst_hist)
    dinv_col = dinv[:NT].reshape(NT, 1)
    dinv2_col = dinv2[:NT].reshape(NT, 1)
    sq4_col = sq4[:NT].reshape(NT, 1)

    all_emb = jnp.concatenate([user_emb, item_emb], axis=0)
    z0 = _tc_scale(all_emb, dinv_col)

    z = z0
    zs = [z0]
    for _ in range(N_LAYERS - 1):
        p = _k1_propagate(z, src_prop, dst_prop)
        z = _tc_combine(p[0], p[1], dinv2_col)
        zs.append(z)

    p = _k1_propagate(z, src_prop, dst_prop)
    out = _tc_final(zs[0], zs[1], zs[2], p[0], p[1], dinv2_col, sq4_col)

    return (out[:N_USERS], out[N_USERS:])


# speedup vs baseline: 6.1912x; 6.1912x over previous
import functools

import jax
import jax.numpy as jnp
from jax import lax
from jax.experimental import pallas as pl
from jax.experimental.pallas import tpu as pltpu
from jax.experimental.pallas import tpu_sc as plsc

N_USERS = 5000
N_ITEMS = 5000
NT = N_USERS + N_ITEMS
D = 128
E = 320000
N_LAYERS = 3
EPS = 1e-7

NC = 2
NS = 16
NW = NC * NS

C = 128
EPW = 10000
CH = 80
NPAD = 10240
RPT = 640

_mesh = plsc.VectorSubcoreMesh(core_axis_name="c", subcore_axis_name="s")


@functools.partial(
    pl.kernel,
    out_type=jax.ShapeDtypeStruct((NC, NPAD, 16), jnp.float32),
    mesh=_mesh,
    scratch_types=[
        pltpu.VMEM((2, 2, C), jnp.int32),
        pltpu.VMEM((C, 16), jnp.float32),
        pltpu.VMEM((C, 16), jnp.float32),
        pltpu.VMEM_SHARED((NPAD, 16), jnp.float32),
        pltpu.SemaphoreType.DMA,
        pltpu.SemaphoreType.DMA,
    ],
)
def _k0_degrees(idx_hbm, deg_hbm, ibuf, ones_v, zrow_v, deg_sp, si0, si1):
    cid = lax.axis_index("c")
    sid = lax.axis_index("s")
    wid = cid * NS + sid
    sis = (si0, si1)

    one_row = jnp.full((16,), 1.0, jnp.float32)
    zrow = jnp.zeros((16,), jnp.float32)

    def init_rows(i, _):
        ones_v[i, :] = one_row
        zrow_v[i, :] = zrow
        return 0
    lax.fori_loop(0, C, init_rows, 0)

    base = sid * RPT
    for k in range(RPT // C):
        pltpu.sync_copy(zrow_v, deg_sp.at[pl.ds(base + k * C, C)])
    plsc.subcore_barrier()

    pltpu.sync_copy(idx_hbm.at[wid, 0], ibuf.at[0])

    def body(jj, _):
        for b in range(2):
            j = jj * 2 + b

            @pl.when(j > 0)
            def _wait():
                pltpu.make_async_copy(
                    idx_hbm.at[wid, j], ibuf.at[b], sis[b]).wait()

            @pl.when(j + 1 < CH)
            def _issue():
                pltpu.async_copy(
                    idx_hbm.at[wid, j + 1], ibuf.at[1 - b], sis[1 - b])

            pltpu.sync_copy(ones_v, deg_sp.at[ibuf.at[b, 1]], add=True)
        return 0

    lax.fori_loop(0, CH // 2, body, 0)
    plsc.subcore_barrier()

    out_base = pl.multiple_of(sid * RPT, 8)
    pltpu.sync_copy(deg_sp.at[pl.ds(out_base, RPT)],
                    deg_hbm.at[cid, pl.ds(out_base, RPT)])


@functools.partial(
    pl.kernel,
    out_type=jax.ShapeDtypeStruct((NC, NT, D), jnp.float32),
    mesh=_mesh,
    scratch_types=[
        pltpu.VMEM((2, 2, C), jnp.int32),
        pltpu.VMEM((2, C, D), jnp.float32),
        pltpu.VMEM_SHARED((NT + 8, D), jnp.float32),
        pltpu.SemaphoreType.DMA,
        pltpu.SemaphoreType.DMA,
        pltpu.SemaphoreType.DMA,
        pltpu.SemaphoreType.DMA,
    ],
)
def _k1_propagate(z_hbm, idx_hbm, out_hbm,
                  ibuf, rows_v, acc_sp, si0, si1, sg0, sg1):
    cid = lax.axis_index("c")
    sid = lax.axis_index("s")
    wid = cid * NS + sid
    sis = (si0, si1)
    sgs = (sg0, sg1)

    zv = jnp.zeros((16,), jnp.float32)

    def zbody(i, _):
        r = i // (D // 16)
        col = (i % (D // 16)) * 16
        rows_v[0, r, pl.ds(col, 16)] = zv
        return 0
    lax.fori_loop(0, C * (D // 16), zbody, 0)

    base = pl.multiple_of(jnp.minimum(sid * RPT, NT - RPT), 8)
    for k in range(RPT // C):
        pltpu.sync_copy(rows_v.at[0], acc_sp.at[pl.ds(base + k * C, C)])
    plsc.subcore_barrier()

    pltpu.sync_copy(idx_hbm.at[wid, 0], ibuf.at[0])
    pltpu.async_copy(z_hbm.at[ibuf.at[0, 0]], rows_v.at[0], sg0)
    pltpu.async_copy(idx_hbm.at[wid, 1], ibuf.at[1], si1)

    def body(jj, _):
        for b in range(2):
            j = jj * 2 + b

            @pl.when(j + 1 < CH)
            def _gather_next():
                pltpu.make_async_copy(
                    idx_hbm.at[wid, j + 1], ibuf.at[1 - b], sis[1 - b]).wait()
                pltpu.async_copy(
                    z_hbm.at[ibuf.at[1 - b, 0]], rows_v.at[1 - b], sgs[1 - b])

            pltpu.make_async_copy(
                z_hbm.at[ibuf.at[b, 0]], rows_v.at[b], sgs[b]).wait()
            pltpu.sync_copy(rows_v.at[b], acc_sp.at[ibuf.at[b, 1]], add=True)

            @pl.when(j + 2 < CH)
            def _prefetch_idx():
                pltpu.async_copy(idx_hbm.at[wid, j + 2], ibuf.at[b], sis[b])
        return 0

    lax.fori_loop(0, CH // 2, body, 0)
    plsc.subcore_barrier()

    pltpu.sync_copy(acc_sp.at[pl.ds(base, RPT)],
                    out_hbm.at[cid, pl.ds(base, RPT)])


_BR = 1000
_row_spec = pl.BlockSpec((_BR, D), lambda i: (i, 0))
_deg_spec = pl.BlockSpec((NC, _BR, 16), lambda i: (0, i, 0))


def _deg_col(deg_ref):
    return deg_ref[0, :, 0:1] + deg_ref[1, :, 0:1] + EPS


def _tc_scale_body(x_ref, deg_ref, o_ref):
    o_ref[...] = x_ref[...] * lax.rsqrt(_deg_col(deg_ref))


_tc_scale = pl.pallas_call(
    _tc_scale_body,
    grid=(NT // _BR,),
    in_specs=[_row_spec, _deg_spec],
    out_specs=_row_spec,
    out_shape=jax.ShapeDtypeStruct((NT, D), jnp.float32),
)


def _tc_combine_body(p0_ref, p1_ref, deg_ref, o_ref):
    o_ref[...] = (p0_ref[...] + p1_ref[...]) / _deg_col(deg_ref)


_tc_combine = pl.pallas_call(
    _tc_combine_body,
    grid=(NT // _BR,),
    in_specs=[_row_spec, _row_spec, _deg_spec],
    out_specs=_row_spec,
    out_shape=jax.ShapeDtypeStruct((NT, D), jnp.float32),
)


def _tc_final_body(z0_ref, z1_ref, z2_ref, p0_ref, p1_ref, deg_ref, o_ref):
    d = _deg_col(deg_ref)
    z3 = (p0_ref[...] + p1_ref[...]) / d
    o_ref[...] = ((z0_ref[...] + z1_ref[...] + z2_ref[...] + z3)
                  * (0.25 * lax.sqrt(d)))


_tc_final = pl.pallas_call(
    _tc_final_body,
    grid=(NT // _BR,),
    in_specs=[_row_spec, _row_spec, _row_spec, _row_spec, _row_spec,
              _deg_spec],
    out_specs=_row_spec,
    out_shape=jax.ShapeDtypeStruct((NT, D), jnp.float32),
)


def kernel(user_emb, item_emb, edge_index):
    src = edge_index[0]
    dst = edge_index[1]

    pad = CH * C - EPW
    src2 = jnp.pad(src.reshape(NW, EPW), ((0, 0), (0, pad)),
                   constant_values=0)
    dst2 = jnp.pad(dst.reshape(NW, EPW), ((0, 0), (0, pad)),
                   constant_values=NT)
    idx = jnp.stack([src2.reshape(NW, CH, C), dst2.reshape(NW, CH, C)],
                    axis=2)

    deg16 = _k0_degrees(idx)[:, :NT]

    all_emb = jnp.concatenate([user_emb, item_emb], axis=0)
    z0 = _tc_scale(all_emb, deg16)

    z = z0
    zs = [z0]
    for _ in range(N_LAYERS - 1):
        p = _k1_propagate(z, idx)
        z = _tc_combine(p[0], p[1], deg16)
        zs.append(z)

    p = _k1_propagate(z, idx)
    out = _tc_final(zs[0], zs[1], zs[2], p[0], p[1], deg16)

    return (out[:N_USERS], out[N_USERS:])
